# baseline (speedup 1.0000x reference)
import functools

import jax
import jax.numpy as jnp
from jax import lax
from jax.experimental import pallas as pl
from jax.experimental.pallas import tpu as pltpu
from jax.experimental.pallas import tpu_sc as plsc


_NC = 2
_NS = 16
_NW = _NC * _NS
_C = 4
_NBUF = 3
_L = 16


def _sc_body(seq_len, batch, d_model, x_hbm, pe_hbm, out_hbm, xv, pev, *sems):
    rows_per_w = seq_len // _NW
    n_chunks = rows_per_w // _C
    n_vec = d_model // _L
    sem_ix = sems[0:_NBUF]
    sem_o = sems[_NBUF:2 * _NBUF]
    sem_pe = sems[2 * _NBUF]
    wid = lax.axis_index("s") * _NC + lax.axis_index("c")
    base = wid * rows_per_w

    cpe = pltpu.async_copy(pe_hbm.at[pl.ds(base, rows_per_w)], pev, sem_pe)

    def start_in(g):
        bi = g % _NBUF
        return pltpu.async_copy(x_hbm.at[pl.ds(base + g * _C, _C)], xv.at[bi],
                                sem_ix[bi])

    def start_out(g):
        bi = g % _NBUF
        return pltpu.async_copy(xv.at[bi], out_hbm.at[pl.ds(base + g * _C, _C)],
                                sem_o[bi])

    def compute(g):
        bi = g % _NBUF
        prow = g * _C

        @plsc.parallel_loop(0, _C * n_vec, unroll=4)
        def _(t):
            s = t // n_vec
            off = (t % n_vec) * _L
            pv = pev[prow + s, pl.ds(off, _L)]
            for b in range(batch):
                xv[bi, s, b, pl.ds(off, _L)] = xv[bi, s, b, pl.ds(off, _L)] + pv

    pend_in = {g: start_in(g) for g in range(min(_NBUF, n_chunks))}
    pend_out = {}
    cpe.wait()
    for g in range(n_chunks):
        if g >= 1 and g + 2 < n_chunks and (g + 2) not in pend_in:
            pend_out.pop(g - 1).wait()
            pend_in[g + 2] = start_in(g + 2)
        pend_in.pop(g).wait()
        compute(g)
        pend_out[g] = start_out(g)
    for g in sorted(pend_out):
        pend_out.pop(g).wait()


def kernel(x, pe):
    seq_len, batch, d_model = x.shape
    mesh = plsc.VectorSubcoreMesh(
        core_axis_name="c", subcore_axis_name="s",
        num_cores=_NC, num_subcores=_NS,
    )
    body = functools.partial(_sc_body, seq_len, batch, d_model)
    return pl.kernel(
        body,
        out_type=jax.ShapeDtypeStruct((seq_len, batch, d_model), x.dtype),
        mesh=mesh,
        scratch_types=[
            pltpu.VMEM((_NBUF, _C, batch, d_model), jnp.float32),
            pltpu.VMEM((seq_len // _NW, d_model), jnp.float32),
        ] + [pltpu.SemaphoreType.DMA] * (2 * _NBUF + 1),
    )(x, pe[:seq_len])


# speedup vs baseline: 1.0546x; 1.0546x over previous
import functools

import jax
import jax.numpy as jnp
from jax import lax
from jax.experimental import pallas as pl
from jax.experimental.pallas import tpu as pltpu
from jax.experimental.pallas import tpu_sc as plsc


_NC = 2
_NS = 16
_NW = _NC * _NS
_C = 4
_NBUF = 4
_L = 16


def _sc_body(seq_len, batch, d_model, x_hbm, pe_hbm, out_hbm, xv, pev, *sems):
    rows_per_w = seq_len // _NW
    n_chunks = rows_per_w // _C
    n_groups = n_chunks // _NBUF
    n_vec = d_model // _L
    sem_ix = sems[0:_NBUF]
    sem_ip = sems[_NBUF:2 * _NBUF]
    sem_o = sems[2 * _NBUF:3 * _NBUF]
    wid = lax.axis_index("s") * _NC + lax.axis_index("c")
    base = wid * rows_per_w

    def in_copies(g, b):
        row = base + g * _C
        return (
            pltpu.make_async_copy(x_hbm.at[pl.ds(row, _C)], xv.at[b],
                                  sem_ix[b]),
            pltpu.make_async_copy(pe_hbm.at[pl.ds(row, _C)], pev.at[b],
                                  sem_ip[b]),
        )

    def out_copy(g, b):
        row = base + g * _C
        return pltpu.make_async_copy(xv.at[b], out_hbm.at[pl.ds(row, _C)],
                                     sem_o[b])

    def start_in(g, b):
        cx, cp = in_copies(g, b)
        cx.start()
        cp.start()

    def compute(b):
        @plsc.parallel_loop(0, _C * n_vec, unroll=4)
        def _(t):
            s = t // n_vec
            off = (t % n_vec) * _L
            pv = pev[b, s, pl.ds(off, _L)]
            for bb in range(batch):
                xv[b, s, bb, pl.ds(off, _L)] = xv[b, s, bb, pl.ds(off, _L)] + pv

    for b in range(_NBUF):
        start_in(b, b)

    def group(k, carry):
        for b in range(_NBUF):
            g = k * _NBUF + b
            cx, cp = in_copies(g, b)
            cx.wait()
            cp.wait()
            compute(b)
            out_copy(g, b).start()
            bp = (b + 2) % _NBUF

            @pl.when(jnp.logical_and(g - 2 >= 0, g + 2 < n_chunks))
            def _():
                out_copy(g - 2, bp).wait()
                start_in(g + 2, bp)
        return carry

    lax.fori_loop(0, n_groups, group, 0, unroll=False)

    for b in range(_NBUF):
        g = (n_groups - 1) * _NBUF + b
        out_copy(g, b).wait()


def kernel(x, pe):
    seq_len, batch, d_model = x.shape
    mesh = plsc.VectorSubcoreMesh(
        core_axis_name="c", subcore_axis_name="s",
        num_cores=_NC, num_subcores=_NS,
    )
    body = functools.partial(_sc_body, seq_len, batch, d_model)
    return pl.kernel(
        body,
        out_type=jax.ShapeDtypeStruct((seq_len, batch, d_model), x.dtype),
        mesh=mesh,
        scratch_types=[
            pltpu.VMEM((_NBUF, _C, batch, d_model), jnp.float32),
            pltpu.VMEM((_NBUF, _C, d_model), jnp.float32),
        ] + [pltpu.SemaphoreType.DMA] * (3 * _NBUF),
    )(x, pe[:seq_len])
